# baseline (speedup 1.0000x reference)
import functools

import jax
import jax.numpy as jnp
from jax import lax
from jax.experimental import pallas as pl
from jax.experimental.pallas import tpu as pltpu
from jax.experimental.pallas import tpu_sc as plsc

_NBUF = 5
_LANES = 16


def _sc_bag_sum(idx2, table, num_cores, num_subcores):
    NW, TW = idx2.shape
    V, D = table.shape
    mesh = plsc.VectorSubcoreMesh(core_axis_name="c", subcore_axis_name="s")

    def build(L):
        bpw = TW // L
        B = NW * bpw
        assert L % _NBUF == 0
        rounds = L // _NBUF
        groups = bpw // _LANES

        @functools.partial(
            pl.kernel,
            mesh=mesh,
            out_type=jax.ShapeDtypeStruct((_NBUF, B, D), jnp.float32),
            scratch_types=[
                pltpu.VMEM((TW,), jnp.int32),
                pltpu.VMEM((L, bpw), jnp.int32),
                pltpu.VMEM((_NBUF, bpw, D), jnp.float32),
            ]
            + [pltpu.SemaphoreType.DMA] * _NBUF,
            compiler_params=pltpu.CompilerParams(
                use_tc_tiling_on_sc=False, needs_layout_passes=False
            ),
        )
        def sc_bag(idx_hbm, table_hbm, sums_hbm, idx_v, idxT_v, acc_v, *sems):
            w = lax.axis_index("s") * num_cores + lax.axis_index("c")
            pltpu.sync_copy(idx_hbm.at[w], idx_v)

            lane_base = lax.iota(jnp.int32, _LANES) * L

            def tr_body(j, carry):
                for g in range(groups):
                    vals = plsc.load_gather(idx_v, [lane_base + (g * _LANES * L + j)])
                    idxT_v[j, pl.ds(g * _LANES, _LANES)] = vals
                return carry

            lax.fori_loop(0, L, tr_body, 0)

            for k in range(_NBUF):
                pltpu.async_copy(table_hbm.at[idxT_v.at[k]], acc_v.at[k], sems[k])

            def round_body(r, carry):
                for k in range(_NBUF):
                    j = r * _NBUF + k
                    pltpu.make_async_copy(
                        table_hbm.at[idxT_v.at[k]], acc_v.at[k], sems[k]
                    ).wait()
                    pltpu.async_copy(
                        table_hbm.at[idxT_v.at[j]], acc_v.at[k], sems[k], add=True
                    )
                return carry

            lax.fori_loop(1, rounds, round_body, 0)

            base = w * bpw
            for k in range(_NBUF):
                pltpu.make_async_copy(
                    table_hbm.at[idxT_v.at[k]], acc_v.at[k], sems[k]
                ).wait()
                pltpu.sync_copy(acc_v.at[k], sums_hbm.at[k].at[pl.ds(base, bpw)])

        return sc_bag

    return build


def kernel(text, offsets, table, W_fc, b_fc):
    T = text.shape[0]
    B = offsets.shape[0]
    L = T // B
    V, D = table.shape
    NCAT = W_fc.shape[0]

    info = plsc.get_sparse_core_info()
    NW = info.num_cores * info.num_subcores
    bpw = B // NW

    idx2 = text.reshape(NW, bpw * L)
    sc_bag = _sc_bag_sum(idx2, table, info.num_cores, info.num_subcores)(L)
    sums = sc_bag(idx2, table)

    inv_l = 1.0 / float(L)

    def mm_body(sums_ref, w_ref, b_ref, out_ref):
        s = jnp.sum(sums_ref[...], axis=0)
        w_scaled = w_ref[...] * inv_l
        out_ref[...] = (
            lax.dot_general(
                s,
                w_scaled,
                (((1,), (1,)), ((), ())),
                preferred_element_type=jnp.float32,
            )
            + b_ref[...]
        )

    out = pl.pallas_call(
        mm_body,
        out_shape=jax.ShapeDtypeStruct((B, NCAT), jnp.float32),
    )(sums, W_fc, b_fc.reshape(1, NCAT))
    return out
